# baseline (speedup 1.0000x reference)
import functools

import jax
import jax.numpy as jnp
from jax import lax
from jax.experimental import pallas as pl
from jax.experimental.pallas import tpu as pltpu
from jax.experimental.pallas import tpu_sc as plsc

N_TOK = 16384
HIDDEN = 2048
E = 16
K = 2
BT = 1024

_SC_INFO = plsc.get_sparse_core_info()
NC = _SC_INFO.num_cores
NS = _SC_INFO.num_subcores
L = _SC_INFO.num_lanes
NW = NC * NS

NCHUNK = 2
TOK_C = N_TOK // NCHUNK
GRID_C = TOK_C // BT
CH = TOK_C // NW
SLABS_PER_STEP = BT // CH


def _gate_kernel(x_ref, w_ref, lg_ref, z_ref):
    i = pl.program_id(0)
    w = w_ref[...]
    logits = lax.dot_general(
        w, x_ref[...], (((1,), (1,)), ((), ())),
        preferred_element_type=jnp.float32,
    )

    part = jnp.sum(logits * logits)

    @pl.when(i == 0)
    def _():
        z_ref[0] = 0.0

    z_ref[0] += part

    for s in range(SLABS_PER_STEP):
        lg_ref[s] = logits[:, s * CH:(s + 1) * CH]


def _route_sc(lg_hbm, i1_hbm, i2_hbm, s1_hbm, s2_hbm,
              buf, oi1, oi2, os1, os2):
    wid = lax.axis_index("s") * NC + lax.axis_index("c")
    pltpu.sync_copy(lg_hbm.at[wid], buf)

    def body(g, carry):
        base = g * L
        neg = jnp.full((L,), -jnp.inf, jnp.float32)
        m1 = neg
        m2 = neg
        i1 = jnp.zeros((L,), jnp.int32)
        i2 = jnp.zeros((L,), jnp.int32)
        for e in range(E):
            v = buf[e, pl.ds(base, L)]
            gt1 = v > m1
            gt2 = v > m2
            m2 = jnp.where(gt1, m1, jnp.where(gt2, v, m2))
            i2 = jnp.where(gt1, i1, jnp.where(gt2, e, i2))
            m1 = jnp.where(gt1, v, m1)
            i1 = jnp.where(gt1, e, i1)
        den = jnp.zeros((L,), jnp.float32)
        for e in range(E):
            v = buf[e, pl.ds(base, L)]
            den = den + jnp.exp(v - m1)
        oi1[pl.ds(base, L)] = i1
        oi2[pl.ds(base, L)] = i2
        os1[pl.ds(base, L)] = 1.0 / den
        os2[pl.ds(base, L)] = jnp.exp(m2 - m1) / den
        return carry

    lax.fori_loop(0, CH // L, body, 0)

    pltpu.sync_copy(oi1, i1_hbm.at[wid])
    pltpu.sync_copy(oi2, i2_hbm.at[wid])
    pltpu.sync_copy(os1, s1_hbm.at[wid])
    pltpu.sync_copy(os2, s2_hbm.at[wid])


_route_call = functools.partial(
    pl.kernel,
    mesh=plsc.VectorSubcoreMesh(core_axis_name="c", subcore_axis_name="s"),
    out_type=[
        jax.ShapeDtypeStruct((NW, CH), jnp.int32),
        jax.ShapeDtypeStruct((NW, CH), jnp.int32),
        jax.ShapeDtypeStruct((NW, CH), jnp.float32),
        jax.ShapeDtypeStruct((NW, CH), jnp.float32),
    ],
    scratch_types=[
        pltpu.VMEM((E, CH), jnp.float32),
        pltpu.VMEM((CH,), jnp.int32),
        pltpu.VMEM((CH,), jnp.int32),
        pltpu.VMEM((CH,), jnp.float32),
        pltpu.VMEM((CH,), jnp.float32),
    ],
)(_route_sc)


def _gate_chunk(c, x, W):
    c0 = c * GRID_C
    return pl.pallas_call(
        _gate_kernel,
        grid=(GRID_C,),
        in_specs=[
            pl.BlockSpec((BT, HIDDEN), lambda i: (c0 + i, 0)),
            pl.BlockSpec((E, HIDDEN), lambda i: (0, 0)),
        ],
        out_specs=[
            pl.BlockSpec((SLABS_PER_STEP, E, CH), lambda i: (i, 0, 0)),
            pl.BlockSpec(memory_space=pltpu.SMEM),
        ],
        out_shape=[
            jax.ShapeDtypeStruct((NW, E, CH), jnp.float32),
            jax.ShapeDtypeStruct((1,), jnp.float32),
        ],
    )(x, W)


def kernel(x, W):
    i1s, i2s, s1s, s2s, zs = [], [], [], [], []
    for c in range(NCHUNK):
        lg, zsum = _gate_chunk(c, x, W)
        i1, i2, s1, s2 = _route_call(lg)
        i1s.append(i1.reshape(TOK_C))
        i2s.append(i2.reshape(TOK_C))
        s1s.append(s1.reshape(TOK_C))
        s2s.append(s2.reshape(TOK_C))
        zs.append(zsum[0])

    idx = jnp.stack([jnp.concatenate(i1s), jnp.concatenate(i2s)], axis=-1)
    scores = jnp.stack([jnp.concatenate(s1s), jnp.concatenate(s2s)], axis=-1)
    z_loss = sum(zs) / jnp.float32(N_TOK * E)
    aux_loss = jnp.zeros((), jnp.float32)
    return (idx, scores, aux_loss, z_loss)


# speedup vs baseline: 1.5607x; 1.5607x over previous
import jax
import jax.numpy as jnp
from jax import lax
from jax.experimental import pallas as pl
from jax.experimental.pallas import tpu as pltpu

N_TOK = 16384
HIDDEN = 2048
E = 16
K = 2
BT = 512
NSTREAM = 2
BTT = BT * NSTREAM
GRID = N_TOK // BTT


def _top2(logits):
    iota = lax.broadcasted_iota(jnp.int32, (E, BT), 0)
    m1 = jnp.max(logits, axis=0, keepdims=True)
    i1 = jnp.min(jnp.where(logits == m1, iota, E), axis=0, keepdims=True)
    l2 = jnp.where(iota == i1, -jnp.inf, logits)
    m2 = jnp.max(l2, axis=0, keepdims=True)
    i2 = jnp.min(jnp.where(l2 == m2, iota, E), axis=0, keepdims=True)
    denom = jnp.sum(jnp.exp(logits - m1), axis=0, keepdims=True)
    s1 = 1.0 / denom
    s2 = jnp.exp(m2 - m1) / denom
    return (jnp.concatenate([i1, i2], axis=0),
            jnp.concatenate([s1, s2], axis=0))


def _router_kernel(xa_ref, xb_ref, w_ref, idx_ref, scr_ref, z_ref):
    i = pl.program_id(0)
    w = w_ref[...]
    la = lax.dot_general(
        w, xa_ref[...], (((1,), (1,)), ((), ())),
        preferred_element_type=jnp.float32,
    )
    lb = lax.dot_general(
        w, xb_ref[...], (((1,), (1,)), ((), ())),
        preferred_element_type=jnp.float32,
    )

    z_ref[0, 0, 0] = jnp.sum(la * la) + jnp.sum(lb * lb)

    ia, sa = _top2(la)
    ib, sb = _top2(lb)
    idx_ref[...] = jnp.concatenate([ia, ib], axis=1)
    scr_ref[...] = jnp.concatenate([sa, sb], axis=1)


def kernel(x, W):
    idx_t, scr_t, zsum = pl.pallas_call(
        _router_kernel,
        grid=(GRID,),
        in_specs=[
            pl.BlockSpec((BT, HIDDEN), lambda i: (2 * i, 0)),
            pl.BlockSpec((BT, HIDDEN), lambda i: (2 * i + 1, 0)),
            pl.BlockSpec((E, HIDDEN), lambda i: (0, 0)),
        ],
        out_specs=[
            pl.BlockSpec((K, BTT), lambda i: (0, i)),
            pl.BlockSpec((K, BTT), lambda i: (0, i)),
            pl.BlockSpec((1, 1, 1), lambda i: (i, 0, 0),
                         memory_space=pltpu.SMEM),
        ],
        out_shape=[
            jax.ShapeDtypeStruct((K, N_TOK), jnp.int32),
            jax.ShapeDtypeStruct((K, N_TOK), jnp.float32),
            jax.ShapeDtypeStruct((GRID, 1, 1), jnp.float32),
        ],
        compiler_params=pltpu.CompilerParams(
            dimension_semantics=("parallel",),
        ),
    )(x, x, W)
    z_loss = jnp.sum(zsum) / jnp.float32(N_TOK * E)
    aux_loss = jnp.zeros((), jnp.float32)
    return (idx_t.T, scr_t.T, aux_loss, z_loss)


# speedup vs baseline: 1.5953x; 1.0221x over previous
import jax
import jax.numpy as jnp
from jax import lax
from jax.experimental import pallas as pl
from jax.experimental.pallas import tpu as pltpu

N_TOK = 16384
HIDDEN = 2048
E = 16
K = 2
BT = 512
NSTREAM = 2
BTT = BT * NSTREAM
GRID = N_TOK // BTT


def _top2(logits):
    iota = lax.broadcasted_iota(jnp.int32, (E, BT), 0)
    m1 = jnp.max(logits, axis=0, keepdims=True)
    i1 = jnp.min(jnp.where(logits == m1, iota, E), axis=0, keepdims=True)
    l2 = jnp.where(iota == i1, -jnp.inf, logits)
    m2 = jnp.max(l2, axis=0, keepdims=True)
    i2 = jnp.min(jnp.where(l2 == m2, iota, E), axis=0, keepdims=True)
    denom = jnp.sum(jnp.exp(logits - m1), axis=0, keepdims=True)
    s1 = 1.0 / denom
    s2 = jnp.exp(m2 - m1) / denom
    return (jnp.concatenate([i1, i2], axis=0),
            jnp.concatenate([s1, s2], axis=0))


def _router_kernel(xa_ref, xb_ref, w_ref, idx_ref, scr_ref, z_ref):
    i = pl.program_id(0)
    w = w_ref[...]
    la = lax.dot_general(
        w, xa_ref[...], (((1,), (1,)), ((), ())),
        preferred_element_type=jnp.float32,
    )
    lb = lax.dot_general(
        w, xb_ref[...], (((1,), (1,)), ((), ())),
        preferred_element_type=jnp.float32,
    )

    part = jnp.sum(la * la) + jnp.sum(lb * lb)

    @pl.when(i == 0)
    def _():
        z_ref[0] = 0.0

    z_ref[0] += part

    ia, sa = _top2(la)
    ib, sb = _top2(lb)
    idx_ref[...] = jnp.concatenate([ia, ib], axis=1)
    scr_ref[...] = jnp.concatenate([sa, sb], axis=1)


def kernel(x, W):
    idx_t, scr_t, zsum = pl.pallas_call(
        _router_kernel,
        grid=(GRID,),
        in_specs=[
            pl.BlockSpec((BT, HIDDEN), lambda i: (2 * i, 0)),
            pl.BlockSpec((BT, HIDDEN), lambda i: (2 * i + 1, 0)),
            pl.BlockSpec((E, HIDDEN), lambda i: (0, 0)),
        ],
        out_specs=[
            pl.BlockSpec((K, BTT), lambda i: (0, i)),
            pl.BlockSpec((K, BTT), lambda i: (0, i)),
            pl.BlockSpec(memory_space=pltpu.SMEM),
        ],
        out_shape=[
            jax.ShapeDtypeStruct((K, N_TOK), jnp.int32),
            jax.ShapeDtypeStruct((K, N_TOK), jnp.float32),
            jax.ShapeDtypeStruct((1,), jnp.float32),
        ],
    )(x, x, W)
    z_loss = zsum[0] / jnp.float32(N_TOK * E)
    aux_loss = jnp.zeros((), jnp.float32)
    return (idx_t.T, scr_t.T, aux_loss, z_loss)
